# baseline (speedup 1.0000x reference)
import functools

import jax
import jax.numpy as jnp
from jax import lax
from jax.experimental import pallas as pl
from jax.experimental.pallas import tpu as pltpu
from jax.experimental.pallas import tpu_sc as plsc

N = 10000
E = 320000
D = 128

NC = 2
NS = 16
CH = 128
NW = NC * NS

N_PAD = 10240
_EPT = -(-E // NW)
_ITERS = -(-_EPT // CH)
_ITERS += _ITERS % 2
EPT = _ITERS * CH
E_PAD = EPT * NW
ROWS_PT = N_PAD // NS

_SC_PARAMS = dict(use_tc_tiling_on_sc=False)


@functools.cache
def _sc_degree_kernel():
    mesh = plsc.VectorSubcoreMesh(core_axis_name="c", subcore_axis_name="s")
    return pl.kernel(
        _sc_degree_body,
        out_type=jax.ShapeDtypeStruct((NC * N_PAD, D), jnp.float32),
        mesh=mesh,
        compiler_params=pltpu.CompilerParams(**_SC_PARAMS),
        scratch_types=[
            pltpu.VMEM((CH,), jnp.int32),
            pltpu.VMEM((CH, D), jnp.float32),
            pltpu.VMEM_SHARED((N_PAD, D), jnp.float32),
        ],
    )


def _sc_degree_body(dst_hbm, out_hbm, dst_v, rows_v, deg_sh):
    c = lax.axis_index("c")
    s = lax.axis_index("s")

    def _setrows(val):
        def _row(r, _):
            def _col(k, __):
                rows_v[r, pl.ds(k * 16, 16)] = jnp.full((16,), val,
                                                        jnp.float32)
                return __
            lax.fori_loop(0, D // 16, _col, None)
            return _
        lax.fori_loop(0, CH, _row, None)

    _setrows(0.0)

    def _zacc(j, _):
        pltpu.sync_copy(rows_v, deg_sh.at[pl.ds(s * ROWS_PT + j * CH, CH)])
        return _
    lax.fori_loop(0, ROWS_PT // CH, _zacc, None)
    _setrows(1.0)
    plsc.subcore_barrier()

    base = (c * NS + s) * EPT

    def _step(e, _):
        off = base + e * CH
        pltpu.sync_copy(dst_hbm.at[pl.ds(off, CH)], dst_v)
        pltpu.sync_copy(rows_v, deg_sh.at[dst_v], add=True)
        return _
    lax.fori_loop(0, _ITERS, _step, None)

    plsc.subcore_barrier()
    pltpu.sync_copy(deg_sh.at[pl.ds(s * ROWS_PT, ROWS_PT)],
                    out_hbm.at[pl.ds(c * N_PAD + s * ROWS_PT, ROWS_PT)])


@functools.cache
def _sc_scatter_kernel():
    mesh = plsc.VectorSubcoreMesh(core_axis_name="c", subcore_axis_name="s")
    return pl.kernel(
        _sc_scatter_body,
        out_type=jax.ShapeDtypeStruct((NC * N_PAD, D), jnp.float32),
        mesh=mesh,
        compiler_params=pltpu.CompilerParams(**_SC_PARAMS),
        scratch_types=[
            pltpu.VMEM((CH,), jnp.int32),
            pltpu.VMEM((CH,), jnp.int32),
            pltpu.VMEM((CH,), jnp.int32),
            pltpu.VMEM((CH,), jnp.int32),
            pltpu.VMEM((CH, D), jnp.float32),
            pltpu.VMEM((CH, D), jnp.float32),
            pltpu.VMEM_SHARED((N_PAD, D), jnp.float32),
            pltpu.SemaphoreType.DMA,
            pltpu.SemaphoreType.DMA,
            pltpu.SemaphoreType.DMA,
            pltpu.SemaphoreType.DMA,
        ],
    )


SPLIT0 = 2 * _ITERS
SPLIT1 = 2 * _ITERS - SPLIT0


def _sc_scatter_body(hs_hbm, src_hbm, dst_hbm, out_hbm, src0, dst0, src1,
                     dst1, rows0, rows1, acc_sh, sem0, sem1, isem0, isem1):
    c = lax.axis_index("c")
    s = lax.axis_index("s")
    my_iters = jnp.where(c == 0, SPLIT0, SPLIT1)
    base = jnp.where(c == 0, s * SPLIT0, NS * SPLIT0 + s * SPLIT1)

    def _zrow(r, _):
        def _zcol(k, __):
            rows0[r, pl.ds(k * 16, 16)] = jnp.zeros((16,), jnp.float32)
            return __
        lax.fori_loop(0, D // 16, _zcol, None)
        return _
    lax.fori_loop(0, CH, _zrow, None)

    def _zacc(j, _):
        pltpu.sync_copy(rows0, acc_sh.at[pl.ds(s * ROWS_PT + j * CH, CH)])
        return _
    lax.fori_loop(0, ROWS_PT // CH, _zacc, None)
    plsc.subcore_barrier()

    def _idx_start(j, sbuf, dbuf, isem):
        off = (base + j) * CH
        pltpu.async_copy(src_hbm.at[pl.ds(off, CH)], sbuf, isem)
        pltpu.async_copy(dst_hbm.at[pl.ds(off, CH)], dbuf, isem)

    def _idx_wait(j, sbuf, dbuf, isem):
        off = (base + j) * CH
        pltpu.make_async_copy(src_hbm.at[pl.ds(off, CH)], sbuf, isem).wait()
        pltpu.make_async_copy(dst_hbm.at[pl.ds(off, CH)], dbuf, isem).wait()

    @pl.when(my_iters > 0)
    def _():
        _idx_start(0, src0, dst0, isem0)
        _idx_start(1, src1, dst1, isem1)
        _idx_wait(0, src0, dst0, isem0)
        pltpu.async_copy(hs_hbm.at[src0], rows0, sem0)

    def _half(j, sbufA, dbufA, rowsA, semA, isemA, sbufB, dbufB, rowsB, semB,
              isemB):
        pltpu.make_async_copy(hs_hbm.at[sbufA], rowsA, semA).wait()

        @pl.when(j + 1 < my_iters)
        def _():
            _idx_wait(j + 1, sbufB, dbufB, isemB)
            pltpu.async_copy(hs_hbm.at[sbufB], rowsB, semB)
        pltpu.sync_copy(rowsA, acc_sh.at[dbufA], add=True)

        @pl.when(j + 2 < my_iters)
        def _():
            _idx_start(j + 2, sbufA, dbufA, isemA)

    def _pair(k, _):
        j = 2 * k
        _half(j, src0, dst0, rows0, sem0, isem0,
              src1, dst1, rows1, sem1, isem1)
        _half(j + 1, src1, dst1, rows1, sem1, isem1,
              src0, dst0, rows0, sem0, isem0)
        return _
    lax.fori_loop(0, my_iters // 2, _pair, None)

    plsc.subcore_barrier()
    pltpu.sync_copy(acc_sh.at[pl.ds(s * ROWS_PT, ROWS_PT)],
                    out_hbm.at[pl.ds(c * N_PAD + s * ROWS_PT, ROWS_PT)])


BM = 1024


def _pre_body(x_ref, w_ref, degp_ref, hs_ref, dinv_ref):
    deg = (jnp.sum(degp_ref[0], axis=1)
           + jnp.sum(degp_ref[1], axis=1)) * (1.0 / D) + 1.0
    dinv = lax.rsqrt(deg)
    h = jnp.dot(x_ref[...], w_ref[...], preferred_element_type=jnp.float32)
    hs_ref[...] = h * dinv[:, None]
    dinv_ref[...] = dinv[:, None]


def _pre(x_pad, w0, degp):
    return pl.pallas_call(
        _pre_body,
        grid=(N_PAD // BM,),
        in_specs=[
            pl.BlockSpec((BM, D), lambda i: (i, 0)),
            pl.BlockSpec((D, D), lambda i: (0, 0)),
            pl.BlockSpec((NC, BM, D), lambda i: (0, i, 0)),
        ],
        out_specs=[
            pl.BlockSpec((BM, D), lambda i: (i, 0)),
            pl.BlockSpec((BM, 1), lambda i: (i, 0)),
        ],
        out_shape=[
            jax.ShapeDtypeStruct((N_PAD, D), jnp.float32),
            jax.ShapeDtypeStruct((N_PAD, 1), jnp.float32),
        ],
    )(x_pad, w0, degp)


def _mid_body(acc_ref, hs_ref, dinv_ref, b_ref, w_ref, out_ref):
    dinv = dinv_ref[...]
    o = (acc_ref[0] + acc_ref[1] + hs_ref[...]) * dinv + b_ref[...]
    x = jnp.maximum(o, 0.0)
    out_ref[...] = jnp.dot(x, w_ref[...],
                           preferred_element_type=jnp.float32) * dinv


def _mid(acc, hs, dinv, b, w_next):
    return pl.pallas_call(
        _mid_body,
        grid=(N_PAD // BM,),
        in_specs=[
            pl.BlockSpec((NC, BM, D), lambda i: (0, i, 0)),
            pl.BlockSpec((BM, D), lambda i: (i, 0)),
            pl.BlockSpec((BM, 1), lambda i: (i, 0)),
            pl.BlockSpec((1, D), lambda i: (0, 0)),
            pl.BlockSpec((D, D), lambda i: (0, 0)),
        ],
        out_specs=pl.BlockSpec((BM, D), lambda i: (i, 0)),
        out_shape=jax.ShapeDtypeStruct((N_PAD, D), jnp.float32),
    )(acc, hs, dinv, b, w_next)


def _post_body(acc_ref, hs_ref, dinv_ref, b_ref, out_ref):
    out_ref[...] = ((acc_ref[0] + acc_ref[1] + hs_ref[...]) * dinv_ref[...]
                    + b_ref[...])


def _post(acc, hs, dinv, b):
    return pl.pallas_call(
        _post_body,
        grid=(N_PAD // BM,),
        in_specs=[
            pl.BlockSpec((NC, BM, D), lambda i: (0, i, 0)),
            pl.BlockSpec((BM, D), lambda i: (i, 0)),
            pl.BlockSpec((BM, 1), lambda i: (i, 0)),
            pl.BlockSpec((1, D), lambda i: (0, 0)),
        ],
        out_specs=pl.BlockSpec((BM, D), lambda i: (i, 0)),
        out_shape=jax.ShapeDtypeStruct((N_PAD, D), jnp.float32),
    )(acc, hs, dinv, b)


def kernel(x, edge_index, W0, b0, W1, b1, W2, b2):
    x_pad = jnp.zeros((N_PAD, D), jnp.float32).at[:N].set(x)
    pad = jnp.full((E_PAD - E,), N, jnp.int32)
    src_p = jnp.concatenate([edge_index[0], pad])
    dst_p = jnp.concatenate([edge_index[1], pad])

    degp = _sc_degree_kernel()(dst_p).reshape(NC, N_PAD, D)
    hs, dinv = _pre(x_pad, W0, degp)
    acc = _sc_scatter_kernel()(hs, src_p, dst_p).reshape(NC, N_PAD, D)
    hs = _mid(acc, hs, dinv, b0.reshape(1, D), W1)
    acc = _sc_scatter_kernel()(hs, src_p, dst_p).reshape(NC, N_PAD, D)
    hs = _mid(acc, hs, dinv, b1.reshape(1, D), W2)
    acc = _sc_scatter_kernel()(hs, src_p, dst_p).reshape(NC, N_PAD, D)
    out = _post(acc, hs, dinv, b2.reshape(1, D))
    return out[:N]


# speedup vs baseline: 1.0519x; 1.0519x over previous
import functools

import jax
import jax.numpy as jnp
from jax import lax
from jax.experimental import pallas as pl
from jax.experimental.pallas import tpu as pltpu
from jax.experimental.pallas import tpu_sc as plsc

N = 10000
E = 320000
D = 128

NC = 2
NS = 16
CH = 128
NW = NC * NS

N_PAD = 10240
_EPT = -(-E // NW)
_ITERS = -(-_EPT // CH)
_ITERS += _ITERS % 2
EPT = _ITERS * CH
E_PAD = EPT * NW
ROWS_PT = N_PAD // NS

_SC_PARAMS = dict(use_tc_tiling_on_sc=False)


@functools.cache
def _sc_degree_kernel():
    mesh = plsc.VectorSubcoreMesh(core_axis_name="c", subcore_axis_name="s")
    return pl.kernel(
        _sc_degree_body,
        out_type=jax.ShapeDtypeStruct((NC * N_PAD, D), jnp.float32),
        mesh=mesh,
        compiler_params=pltpu.CompilerParams(**_SC_PARAMS),
        scratch_types=[
            pltpu.VMEM((CH,), jnp.int32),
            pltpu.VMEM((CH, D), jnp.float32),
            pltpu.VMEM_SHARED((N_PAD, D), jnp.float32),
        ],
    )


def _sc_degree_body(dst_hbm, out_hbm, dst_v, rows_v, deg_sh):
    c = lax.axis_index("c")
    s = lax.axis_index("s")

    def _setrows(val):
        def _row(r, _):
            def _col(k, __):
                rows_v[r, pl.ds(k * 16, 16)] = jnp.full((16,), val,
                                                        jnp.float32)
                return __
            lax.fori_loop(0, D // 16, _col, None)
            return _
        lax.fori_loop(0, CH, _row, None)

    _setrows(0.0)

    def _zacc(j, _):
        pltpu.sync_copy(rows_v, deg_sh.at[pl.ds(s * ROWS_PT + j * CH, CH)])
        return _
    lax.fori_loop(0, ROWS_PT // CH, _zacc, None)
    _setrows(1.0)
    plsc.subcore_barrier()

    base = (c * NS + s) * EPT

    def _step(e, _):
        off = base + e * CH
        pltpu.sync_copy(dst_hbm.at[pl.ds(off, CH)], dst_v)
        pltpu.sync_copy(rows_v, deg_sh.at[dst_v], add=True)
        return _
    lax.fori_loop(0, _ITERS, _step, None)

    plsc.subcore_barrier()
    pltpu.sync_copy(deg_sh.at[pl.ds(s * ROWS_PT, ROWS_PT)],
                    out_hbm.at[pl.ds(c * N_PAD + s * ROWS_PT, ROWS_PT)])


@functools.cache
def _sc_scatter_kernel():
    mesh = plsc.VectorSubcoreMesh(core_axis_name="c", subcore_axis_name="s")
    return pl.kernel(
        _sc_scatter_body,
        out_type=jax.ShapeDtypeStruct((NC * N_PAD, D), jnp.float32),
        mesh=mesh,
        compiler_params=pltpu.CompilerParams(**_SC_PARAMS),
        scratch_types=[
            pltpu.VMEM((CH,), jnp.int32),
            pltpu.VMEM((CH,), jnp.int32),
            pltpu.VMEM((CH,), jnp.int32),
            pltpu.VMEM((CH,), jnp.int32),
            pltpu.VMEM((CH, D), jnp.float32),
            pltpu.VMEM((CH, D), jnp.float32),
            pltpu.VMEM_SHARED((N_PAD, D), jnp.float32),
            pltpu.SemaphoreType.DMA,
            pltpu.SemaphoreType.DMA,
            pltpu.SemaphoreType.DMA,
            pltpu.SemaphoreType.DMA,
        ],
    )


SPLIT0 = 0
SPLIT1 = 2 * _ITERS - SPLIT0


def _sc_scatter_body(hs_hbm, src_hbm, dst_hbm, out_hbm, src0, dst0, src1,
                     dst1, rows0, rows1, acc_sh, sem0, sem1, isem0, isem1):
    c = lax.axis_index("c")
    s = lax.axis_index("s")
    my_iters = jnp.where(c == 0, SPLIT0, SPLIT1)
    base = jnp.where(c == 0, s * SPLIT0, NS * SPLIT0 + s * SPLIT1)

    def _zrow(r, _):
        def _zcol(k, __):
            rows0[r, pl.ds(k * 16, 16)] = jnp.zeros((16,), jnp.float32)
            return __
        lax.fori_loop(0, D // 16, _zcol, None)
        return _
    lax.fori_loop(0, CH, _zrow, None)

    def _zacc(j, _):
        pltpu.sync_copy(rows0, acc_sh.at[pl.ds(s * ROWS_PT + j * CH, CH)])
        return _
    lax.fori_loop(0, ROWS_PT // CH, _zacc, None)
    plsc.subcore_barrier()

    def _idx_start(j, sbuf, dbuf, isem):
        off = (base + j) * CH
        pltpu.async_copy(src_hbm.at[pl.ds(off, CH)], sbuf, isem)
        pltpu.async_copy(dst_hbm.at[pl.ds(off, CH)], dbuf, isem)

    def _idx_wait(j, sbuf, dbuf, isem):
        off = (base + j) * CH
        pltpu.make_async_copy(src_hbm.at[pl.ds(off, CH)], sbuf, isem).wait()
        pltpu.make_async_copy(dst_hbm.at[pl.ds(off, CH)], dbuf, isem).wait()

    @pl.when(my_iters > 0)
    def _():
        _idx_start(0, src0, dst0, isem0)
        _idx_start(1, src1, dst1, isem1)
        _idx_wait(0, src0, dst0, isem0)
        pltpu.async_copy(hs_hbm.at[src0], rows0, sem0)

    def _half(j, sbufA, dbufA, rowsA, semA, isemA, sbufB, dbufB, rowsB, semB,
              isemB):
        pltpu.make_async_copy(hs_hbm.at[sbufA], rowsA, semA).wait()

        @pl.when(j + 1 < my_iters)
        def _():
            _idx_wait(j + 1, sbufB, dbufB, isemB)
            pltpu.async_copy(hs_hbm.at[sbufB], rowsB, semB)
        pltpu.sync_copy(rowsA, acc_sh.at[dbufA], add=True)

        @pl.when(j + 2 < my_iters)
        def _():
            _idx_start(j + 2, sbufA, dbufA, isemA)

    def _pair(k, _):
        j = 2 * k
        _half(j, src0, dst0, rows0, sem0, isem0,
              src1, dst1, rows1, sem1, isem1)
        _half(j + 1, src1, dst1, rows1, sem1, isem1,
              src0, dst0, rows0, sem0, isem0)
        return _
    lax.fori_loop(0, my_iters // 2, _pair, None)

    plsc.subcore_barrier()
    pltpu.sync_copy(acc_sh.at[pl.ds(s * ROWS_PT, ROWS_PT)],
                    out_hbm.at[pl.ds(c * N_PAD + s * ROWS_PT, ROWS_PT)])


BM = 1024


def _pre_body(x_ref, w_ref, degp_ref, hs_ref, dinv_ref):
    deg = (jnp.sum(degp_ref[0], axis=1)
           + jnp.sum(degp_ref[1], axis=1)) * (1.0 / D) + 1.0
    dinv = lax.rsqrt(deg)
    h = jnp.dot(x_ref[...], w_ref[...], preferred_element_type=jnp.float32)
    hs_ref[...] = h * dinv[:, None]
    dinv_ref[...] = dinv[:, None]


def _pre(x_pad, w0, degp):
    return pl.pallas_call(
        _pre_body,
        grid=(N_PAD // BM,),
        in_specs=[
            pl.BlockSpec((BM, D), lambda i: (i, 0)),
            pl.BlockSpec((D, D), lambda i: (0, 0)),
            pl.BlockSpec((NC, BM, D), lambda i: (0, i, 0)),
        ],
        out_specs=[
            pl.BlockSpec((BM, D), lambda i: (i, 0)),
            pl.BlockSpec((BM, 1), lambda i: (i, 0)),
        ],
        out_shape=[
            jax.ShapeDtypeStruct((N_PAD, D), jnp.float32),
            jax.ShapeDtypeStruct((N_PAD, 1), jnp.float32),
        ],
    )(x_pad, w0, degp)


def _mid_body(acc_ref, hs_ref, dinv_ref, b_ref, w_ref, out_ref):
    dinv = dinv_ref[...]
    o = (acc_ref[0] + acc_ref[1] + hs_ref[...]) * dinv + b_ref[...]
    x = jnp.maximum(o, 0.0)
    out_ref[...] = jnp.dot(x, w_ref[...],
                           preferred_element_type=jnp.float32) * dinv


def _mid(acc, hs, dinv, b, w_next):
    return pl.pallas_call(
        _mid_body,
        grid=(N_PAD // BM,),
        in_specs=[
            pl.BlockSpec((NC, BM, D), lambda i: (0, i, 0)),
            pl.BlockSpec((BM, D), lambda i: (i, 0)),
            pl.BlockSpec((BM, 1), lambda i: (i, 0)),
            pl.BlockSpec((1, D), lambda i: (0, 0)),
            pl.BlockSpec((D, D), lambda i: (0, 0)),
        ],
        out_specs=pl.BlockSpec((BM, D), lambda i: (i, 0)),
        out_shape=jax.ShapeDtypeStruct((N_PAD, D), jnp.float32),
    )(acc, hs, dinv, b, w_next)


def _post_body(acc_ref, hs_ref, dinv_ref, b_ref, out_ref):
    out_ref[...] = ((acc_ref[0] + acc_ref[1] + hs_ref[...]) * dinv_ref[...]
                    + b_ref[...])


def _post(acc, hs, dinv, b):
    return pl.pallas_call(
        _post_body,
        grid=(N_PAD // BM,),
        in_specs=[
            pl.BlockSpec((NC, BM, D), lambda i: (0, i, 0)),
            pl.BlockSpec((BM, D), lambda i: (i, 0)),
            pl.BlockSpec((BM, 1), lambda i: (i, 0)),
            pl.BlockSpec((1, D), lambda i: (0, 0)),
        ],
        out_specs=pl.BlockSpec((BM, D), lambda i: (i, 0)),
        out_shape=jax.ShapeDtypeStruct((N_PAD, D), jnp.float32),
    )(acc, hs, dinv, b)


def kernel(x, edge_index, W0, b0, W1, b1, W2, b2):
    x_pad = jnp.zeros((N_PAD, D), jnp.float32).at[:N].set(x)
    pad = jnp.full((E_PAD - E,), N, jnp.int32)
    src_p = jnp.concatenate([edge_index[0], pad])
    dst_p = jnp.concatenate([edge_index[1], pad])

    degp = _sc_degree_kernel()(dst_p).reshape(NC, N_PAD, D)
    hs, dinv = _pre(x_pad, W0, degp)
    acc = _sc_scatter_kernel()(hs, src_p, dst_p).reshape(NC, N_PAD, D)
    hs = _mid(acc, hs, dinv, b0.reshape(1, D), W1)
    acc = _sc_scatter_kernel()(hs, src_p, dst_p).reshape(NC, N_PAD, D)
    hs = _mid(acc, hs, dinv, b1.reshape(1, D), W2)
    acc = _sc_scatter_kernel()(hs, src_p, dst_p).reshape(NC, N_PAD, D)
    out = _post(acc, hs, dinv, b2.reshape(1, D))
    return out[:N]


# speedup vs baseline: 2.6368x; 2.5068x over previous
import functools

import jax
import jax.numpy as jnp
from jax import lax
from jax.experimental import pallas as pl
from jax.experimental.pallas import tpu as pltpu
from jax.experimental.pallas import tpu_sc as plsc

N = 10000
E = 320000
D = 128

NC = 2
NS = 16
CH = 128
NW = NC * NS

N_PAD = 10240
_EPT = -(-E // NW)
_ITERS = -(-_EPT // CH)
_ITERS += _ITERS % 2
EPT = _ITERS * CH
E_PAD = EPT * NW
ROWS_PT = N_PAD // NS

_SC_PARAMS = dict(use_tc_tiling_on_sc=False)


@functools.cache
def _sc_degree_kernel():
    mesh = plsc.VectorSubcoreMesh(core_axis_name="c", subcore_axis_name="s")
    return pl.kernel(
        _sc_degree_body,
        out_type=jax.ShapeDtypeStruct((NC * N_PAD, D), jnp.float32),
        mesh=mesh,
        compiler_params=pltpu.CompilerParams(**_SC_PARAMS),
        scratch_types=[
            pltpu.VMEM((CH,), jnp.int32),
            pltpu.VMEM((CH, D), jnp.float32),
            pltpu.VMEM_SHARED((N_PAD, D), jnp.float32),
        ],
    )


def _sc_degree_body(dst_hbm, out_hbm, dst_v, rows_v, deg_sh):
    c = lax.axis_index("c")
    s = lax.axis_index("s")

    def _setrows(val):
        def _row(r, _):
            def _col(k, __):
                rows_v[r, pl.ds(k * 16, 16)] = jnp.full((16,), val,
                                                        jnp.float32)
                return __
            lax.fori_loop(0, D // 16, _col, None)
            return _
        lax.fori_loop(0, CH, _row, None)

    _setrows(0.0)

    def _zacc(j, _):
        pltpu.sync_copy(rows_v, deg_sh.at[pl.ds(s * ROWS_PT + j * CH, CH)])
        return _
    lax.fori_loop(0, ROWS_PT // CH, _zacc, None)
    _setrows(1.0)
    plsc.subcore_barrier()

    base = (c * NS + s) * EPT

    def _step(e, _):
        off = base + e * CH
        pltpu.sync_copy(dst_hbm.at[pl.ds(off, CH)], dst_v)
        pltpu.sync_copy(rows_v, deg_sh.at[dst_v], add=True)
        return _
    lax.fori_loop(0, _ITERS, _step, None)

    plsc.subcore_barrier()
    pltpu.sync_copy(deg_sh.at[pl.ds(s * ROWS_PT, ROWS_PT)],
                    out_hbm.at[pl.ds(c * N_PAD + s * ROWS_PT, ROWS_PT)])


DH = D // NC
ITERS_ALL = E_PAD // CH // NS


@functools.cache
def _sc_scatter_kernel():
    mesh = plsc.VectorSubcoreMesh(core_axis_name="c", subcore_axis_name="s")
    return pl.kernel(
        _sc_scatter_body,
        out_type=jax.ShapeDtypeStruct((N_PAD, D), jnp.float32),
        mesh=mesh,
        compiler_params=pltpu.CompilerParams(**_SC_PARAMS),
        scratch_types=[
            pltpu.VMEM((CH,), jnp.int32),
            pltpu.VMEM((CH,), jnp.int32),
            pltpu.VMEM((CH,), jnp.int32),
            pltpu.VMEM((CH,), jnp.int32),
            pltpu.VMEM((CH, DH), jnp.float32),
            pltpu.VMEM((CH, DH), jnp.float32),
            pltpu.VMEM_SHARED((N_PAD, DH), jnp.float32),
            pltpu.VMEM_SHARED((N_PAD, DH), jnp.float32),
            pltpu.SemaphoreType.DMA,
            pltpu.SemaphoreType.DMA,
            pltpu.SemaphoreType.DMA,
            pltpu.SemaphoreType.DMA,
        ],
    )


def _sc_scatter_body(hs_hbm, src_hbm, dst_hbm, out_hbm, src0, dst0, src1,
                     dst1, rows0, rows1, hs_sh, acc_sh, sem0, sem1, isem0,
                     isem1):
    c = lax.axis_index("c")
    s = lax.axis_index("s")
    base = s * ITERS_ALL

    pltpu.sync_copy(hs_hbm.at[pl.ds(s * ROWS_PT, ROWS_PT),
                              pl.ds(c * DH, DH)],
                    hs_sh.at[pl.ds(s * ROWS_PT, ROWS_PT)])

    def _zrow(r, _):
        def _zcol(k, __):
            rows0[r, pl.ds(k * 16, 16)] = jnp.zeros((16,), jnp.float32)
            return __
        lax.fori_loop(0, DH // 16, _zcol, None)
        return _
    lax.fori_loop(0, CH, _zrow, None)

    def _zacc(j, _):
        pltpu.sync_copy(rows0, acc_sh.at[pl.ds(s * ROWS_PT + j * CH, CH)])
        return _
    lax.fori_loop(0, ROWS_PT // CH, _zacc, None)
    plsc.subcore_barrier()

    def _idx_start(j, sbuf, dbuf, isem):
        off = (base + j) * CH
        pltpu.async_copy(src_hbm.at[pl.ds(off, CH)], sbuf, isem)
        pltpu.async_copy(dst_hbm.at[pl.ds(off, CH)], dbuf, isem)

    def _idx_wait(j, sbuf, dbuf, isem):
        off = (base + j) * CH
        pltpu.make_async_copy(src_hbm.at[pl.ds(off, CH)], sbuf, isem).wait()
        pltpu.make_async_copy(dst_hbm.at[pl.ds(off, CH)], dbuf, isem).wait()

    _idx_start(0, src0, dst0, isem0)
    _idx_start(1, src1, dst1, isem1)
    _idx_wait(0, src0, dst0, isem0)
    pltpu.async_copy(hs_sh.at[src0], rows0, sem0)

    def _half(j, sbufA, dbufA, rowsA, semA, isemA, sbufB, dbufB, rowsB, semB,
              isemB):
        pltpu.make_async_copy(hs_sh.at[sbufA], rowsA, semA).wait()

        @pl.when(j + 1 < ITERS_ALL)
        def _():
            _idx_wait(j + 1, sbufB, dbufB, isemB)
            pltpu.async_copy(hs_sh.at[sbufB], rowsB, semB)
        pltpu.sync_copy(rowsA, acc_sh.at[dbufA], add=True)

        @pl.when(j + 2 < ITERS_ALL)
        def _():
            _idx_start(j + 2, sbufA, dbufA, isemA)

    def _pair(k, _):
        j = 2 * k
        _half(j, src0, dst0, rows0, sem0, isem0,
              src1, dst1, rows1, sem1, isem1)
        _half(j + 1, src1, dst1, rows1, sem1, isem1,
              src0, dst0, rows0, sem0, isem0)
        return _
    lax.fori_loop(0, ITERS_ALL // 2, _pair, None)

    plsc.subcore_barrier()
    pltpu.sync_copy(acc_sh.at[pl.ds(s * ROWS_PT, ROWS_PT)],
                    out_hbm.at[pl.ds(s * ROWS_PT, ROWS_PT),
                               pl.ds(c * DH, DH)])


BM = 1024


def _pre_body(x_ref, w_ref, degp_ref, hs_ref, dinv_ref):
    deg = (jnp.sum(degp_ref[0], axis=1)
           + jnp.sum(degp_ref[1], axis=1)) * (1.0 / D) + 1.0
    dinv = lax.rsqrt(deg)
    h = jnp.dot(x_ref[...], w_ref[...], preferred_element_type=jnp.float32)
    hs_ref[...] = h * dinv[:, None]
    dinv_ref[...] = dinv[:, None]


def _pre(x_pad, w0, degp):
    return pl.pallas_call(
        _pre_body,
        grid=(N_PAD // BM,),
        in_specs=[
            pl.BlockSpec((BM, D), lambda i: (i, 0)),
            pl.BlockSpec((D, D), lambda i: (0, 0)),
            pl.BlockSpec((NC, BM, D), lambda i: (0, i, 0)),
        ],
        out_specs=[
            pl.BlockSpec((BM, D), lambda i: (i, 0)),
            pl.BlockSpec((BM, 1), lambda i: (i, 0)),
        ],
        out_shape=[
            jax.ShapeDtypeStruct((N_PAD, D), jnp.float32),
            jax.ShapeDtypeStruct((N_PAD, 1), jnp.float32),
        ],
    )(x_pad, w0, degp)


def _mid_body(acc_ref, hs_ref, dinv_ref, b_ref, w_ref, out_ref):
    dinv = dinv_ref[...]
    o = (acc_ref[...] + hs_ref[...]) * dinv + b_ref[...]
    x = jnp.maximum(o, 0.0)
    out_ref[...] = jnp.dot(x, w_ref[...],
                           preferred_element_type=jnp.float32) * dinv


def _mid(acc, hs, dinv, b, w_next):
    return pl.pallas_call(
        _mid_body,
        grid=(N_PAD // BM,),
        in_specs=[
            pl.BlockSpec((BM, D), lambda i: (i, 0)),
            pl.BlockSpec((BM, D), lambda i: (i, 0)),
            pl.BlockSpec((BM, 1), lambda i: (i, 0)),
            pl.BlockSpec((1, D), lambda i: (0, 0)),
            pl.BlockSpec((D, D), lambda i: (0, 0)),
        ],
        out_specs=pl.BlockSpec((BM, D), lambda i: (i, 0)),
        out_shape=jax.ShapeDtypeStruct((N_PAD, D), jnp.float32),
    )(acc, hs, dinv, b, w_next)


def _post_body(acc_ref, hs_ref, dinv_ref, b_ref, out_ref):
    out_ref[...] = ((acc_ref[...] + hs_ref[...]) * dinv_ref[...]
                    + b_ref[...])


def _post(acc, hs, dinv, b):
    return pl.pallas_call(
        _post_body,
        grid=(N_PAD // BM,),
        in_specs=[
            pl.BlockSpec((BM, D), lambda i: (i, 0)),
            pl.BlockSpec((BM, D), lambda i: (i, 0)),
            pl.BlockSpec((BM, 1), lambda i: (i, 0)),
            pl.BlockSpec((1, D), lambda i: (0, 0)),
        ],
        out_specs=pl.BlockSpec((BM, D), lambda i: (i, 0)),
        out_shape=jax.ShapeDtypeStruct((N_PAD, D), jnp.float32),
    )(acc, hs, dinv, b)


def kernel(x, edge_index, W0, b0, W1, b1, W2, b2):
    x_pad = jnp.zeros((N_PAD, D), jnp.float32).at[:N].set(x)
    pad = jnp.full((E_PAD - E,), N, jnp.int32)
    src_p = jnp.concatenate([edge_index[0], pad])
    dst_p = jnp.concatenate([edge_index[1], pad])

    degp = _sc_degree_kernel()(dst_p).reshape(NC, N_PAD, D)
    hs, dinv = _pre(x_pad, W0, degp)
    acc = _sc_scatter_kernel()(hs, src_p, dst_p)
    hs = _mid(acc, hs, dinv, b0.reshape(1, D), W1)
    acc = _sc_scatter_kernel()(hs, src_p, dst_p)
    hs = _mid(acc, hs, dinv, b1.reshape(1, D), W2)
    acc = _sc_scatter_kernel()(hs, src_p, dst_p)
    out = _post(acc, hs, dinv, b2.reshape(1, D))
    return out[:N]


# speedup vs baseline: 2.8627x; 1.0857x over previous
import functools

import jax
import jax.numpy as jnp
from jax import lax
from jax.experimental import pallas as pl
from jax.experimental.pallas import tpu as pltpu
from jax.experimental.pallas import tpu_sc as plsc

N = 10000
E = 320000
D = 128

NC = 2
NS = 16
CH = 128
NW = NC * NS

N_PAD = 10240
_ITERS_ALL = -(-E // (NS * CH))
_ITERS_ALL += _ITERS_ALL % 2
E_PAD = _ITERS_ALL * NS * CH
EPT = E_PAD // NW
_ITERS_DEG = EPT // CH
ROWS_PT = N_PAD // NS

_SC_PARAMS = dict(use_tc_tiling_on_sc=False)


@functools.cache
def _sc_degree_kernel():
    mesh = plsc.VectorSubcoreMesh(core_axis_name="c", subcore_axis_name="s")
    return pl.kernel(
        _sc_degree_body,
        out_type=jax.ShapeDtypeStruct((NC * N_PAD, D), jnp.float32),
        mesh=mesh,
        compiler_params=pltpu.CompilerParams(**_SC_PARAMS),
        scratch_types=[
            pltpu.VMEM((CH,), jnp.int32),
            pltpu.VMEM((CH,), jnp.int32),
            pltpu.VMEM((CH, 16), jnp.float32),
            pltpu.VMEM((ROWS_PT, 16), jnp.float32),
            pltpu.VMEM_SHARED((N_PAD, 16), jnp.float32),
            pltpu.SemaphoreType.DMA,
            pltpu.SemaphoreType.DMA,
        ],
    )


def _sc_degree_body(dst_hbm, out_hbm, dst0, dst1, ones_v, zero_v, deg_sh,
                    isem0, isem1):
    c = lax.axis_index("c")
    s = lax.axis_index("s")
    base = (c * NS + s) * _ITERS_DEG

    def _fill(i, _):
        ones_v[i] = jnp.ones((16,), jnp.float32)
        return _
    lax.fori_loop(0, CH, _fill, None)

    def _zero(i, _):
        zero_v[i] = jnp.zeros((16,), jnp.float32)
        return _
    lax.fori_loop(0, ROWS_PT, _zero, None)

    pltpu.sync_copy(zero_v, deg_sh.at[pl.ds(s * ROWS_PT, ROWS_PT)])
    plsc.subcore_barrier()

    def _idx_start(j, buf, isem):
        pltpu.async_copy(dst_hbm.at[pl.ds((base + j) * CH, CH)], buf, isem)

    def _idx_wait(j, buf, isem):
        pltpu.make_async_copy(dst_hbm.at[pl.ds((base + j) * CH, CH)], buf,
                              isem).wait()

    _idx_start(0, dst0, isem0)

    def _step(e, _):
        def _one(j, buf, isem, nbuf, nisem):
            _idx_wait(j, buf, isem)

            @pl.when(j + 1 < _ITERS_DEG)
            def _():
                _idx_start(j + 1, nbuf, nisem)
            pltpu.sync_copy(ones_v, deg_sh.at[buf], add=True)

        _one(2 * e, dst0, isem0, dst1, isem1)

        @pl.when(2 * e + 1 < _ITERS_DEG)
        def _():
            _one(2 * e + 1, dst1, isem1, dst0, isem0)
        return _
    lax.fori_loop(0, (_ITERS_DEG + 1) // 2, _step, None)

    plsc.subcore_barrier()
    pltpu.sync_copy(deg_sh.at[pl.ds(s * ROWS_PT, ROWS_PT)],
                    out_hbm.at[pl.ds(c * N_PAD + s * ROWS_PT, ROWS_PT),
                               pl.ds(0, 16)])


DH = D // NC
ITERS_ALL = E_PAD // CH // NS


@functools.cache
def _sc_scatter_kernel():
    mesh = plsc.VectorSubcoreMesh(core_axis_name="c", subcore_axis_name="s")
    return pl.kernel(
        _sc_scatter_body,
        out_type=jax.ShapeDtypeStruct((N_PAD, D), jnp.float32),
        mesh=mesh,
        compiler_params=pltpu.CompilerParams(**_SC_PARAMS),
        scratch_types=[
            pltpu.VMEM((CH,), jnp.int32),
            pltpu.VMEM((CH,), jnp.int32),
            pltpu.VMEM((CH,), jnp.int32),
            pltpu.VMEM((CH,), jnp.int32),
            pltpu.VMEM((CH, DH), jnp.float32),
            pltpu.VMEM((CH, DH), jnp.float32),
            pltpu.VMEM_SHARED((N_PAD, DH), jnp.float32),
            pltpu.VMEM_SHARED((N_PAD, DH), jnp.float32),
            pltpu.SemaphoreType.DMA,
            pltpu.SemaphoreType.DMA,
            pltpu.SemaphoreType.DMA,
            pltpu.SemaphoreType.DMA,
        ],
    )


def _sc_scatter_body(hs_hbm, src_hbm, dst_hbm, out_hbm, src0, dst0, src1,
                     dst1, rows0, rows1, hs_sh, acc_sh, sem0, sem1, isem0,
                     isem1):
    c = lax.axis_index("c")
    s = lax.axis_index("s")
    base = s * ITERS_ALL

    pltpu.sync_copy(hs_hbm.at[pl.ds(s * ROWS_PT, ROWS_PT),
                              pl.ds(c * DH, DH)],
                    hs_sh.at[pl.ds(s * ROWS_PT, ROWS_PT)])

    def _zrow(r, _):
        def _zcol(k, __):
            rows0[r, pl.ds(k * 16, 16)] = jnp.zeros((16,), jnp.float32)
            return __
        lax.fori_loop(0, DH // 16, _zcol, None)
        return _
    lax.fori_loop(0, CH, _zrow, None)

    def _zacc(j, _):
        pltpu.sync_copy(rows0, acc_sh.at[pl.ds(s * ROWS_PT + j * CH, CH)])
        return _
    lax.fori_loop(0, ROWS_PT // CH, _zacc, None)
    plsc.subcore_barrier()

    def _idx_start(j, sbuf, dbuf, isem):
        off = (base + j) * CH
        pltpu.async_copy(src_hbm.at[pl.ds(off, CH)], sbuf, isem)
        pltpu.async_copy(dst_hbm.at[pl.ds(off, CH)], dbuf, isem)

    def _idx_wait(j, sbuf, dbuf, isem):
        off = (base + j) * CH
        pltpu.make_async_copy(src_hbm.at[pl.ds(off, CH)], sbuf, isem).wait()
        pltpu.make_async_copy(dst_hbm.at[pl.ds(off, CH)], dbuf, isem).wait()

    _idx_start(0, src0, dst0, isem0)
    _idx_start(1, src1, dst1, isem1)
    _idx_wait(0, src0, dst0, isem0)
    pltpu.async_copy(hs_sh.at[src0], rows0, sem0)

    def _half(j, sbufA, dbufA, rowsA, semA, isemA, sbufB, dbufB, rowsB, semB,
              isemB):
        pltpu.make_async_copy(hs_sh.at[sbufA], rowsA, semA).wait()

        @pl.when(j + 1 < ITERS_ALL)
        def _():
            _idx_wait(j + 1, sbufB, dbufB, isemB)
            pltpu.async_copy(hs_sh.at[sbufB], rowsB, semB)
        pltpu.sync_copy(rowsA, acc_sh.at[dbufA], add=True)

        @pl.when(j + 2 < ITERS_ALL)
        def _():
            _idx_start(j + 2, sbufA, dbufA, isemA)

    def _pair(k, _):
        j = 2 * k
        _half(j, src0, dst0, rows0, sem0, isem0,
              src1, dst1, rows1, sem1, isem1)
        _half(j + 1, src1, dst1, rows1, sem1, isem1,
              src0, dst0, rows0, sem0, isem0)
        return _
    lax.fori_loop(0, ITERS_ALL // 2, _pair, None)

    plsc.subcore_barrier()
    pltpu.sync_copy(acc_sh.at[pl.ds(s * ROWS_PT, ROWS_PT)],
                    out_hbm.at[pl.ds(s * ROWS_PT, ROWS_PT),
                               pl.ds(c * DH, DH)])


BM = 1024


def _pre_body(x_ref, w_ref, degp_ref, hs_ref, dinv_ref):
    deg = (jnp.sum(degp_ref[0], axis=1)
           + jnp.sum(degp_ref[1], axis=1)) * (1.0 / 16.0) + 1.0
    dinv = lax.rsqrt(deg)
    h = jnp.dot(x_ref[...], w_ref[...], preferred_element_type=jnp.float32)
    hs_ref[...] = h * dinv[:, None]
    dinv_ref[...] = dinv[:, None]


def _pre(x_pad, w0, degp):
    return pl.pallas_call(
        _pre_body,
        grid=(N_PAD // BM,),
        in_specs=[
            pl.BlockSpec((BM, D), lambda i: (i, 0)),
            pl.BlockSpec((D, D), lambda i: (0, 0)),
            pl.BlockSpec((NC, BM, 16), lambda i: (0, i, 0)),
        ],
        out_specs=[
            pl.BlockSpec((BM, D), lambda i: (i, 0)),
            pl.BlockSpec((BM, 1), lambda i: (i, 0)),
        ],
        out_shape=[
            jax.ShapeDtypeStruct((N_PAD, D), jnp.float32),
            jax.ShapeDtypeStruct((N_PAD, 1), jnp.float32),
        ],
    )(x_pad, w0, degp)


def _mid_body(acc_ref, hs_ref, dinv_ref, b_ref, w_ref, out_ref):
    dinv = dinv_ref[...]
    o = (acc_ref[...] + hs_ref[...]) * dinv + b_ref[...]
    x = jnp.maximum(o, 0.0)
    out_ref[...] = jnp.dot(x, w_ref[...],
                           preferred_element_type=jnp.float32) * dinv


def _mid(acc, hs, dinv, b, w_next):
    return pl.pallas_call(
        _mid_body,
        grid=(N_PAD // BM,),
        in_specs=[
            pl.BlockSpec((BM, D), lambda i: (i, 0)),
            pl.BlockSpec((BM, D), lambda i: (i, 0)),
            pl.BlockSpec((BM, 1), lambda i: (i, 0)),
            pl.BlockSpec((1, D), lambda i: (0, 0)),
            pl.BlockSpec((D, D), lambda i: (0, 0)),
        ],
        out_specs=pl.BlockSpec((BM, D), lambda i: (i, 0)),
        out_shape=jax.ShapeDtypeStruct((N_PAD, D), jnp.float32),
    )(acc, hs, dinv, b, w_next)


def _post_body(acc_ref, hs_ref, dinv_ref, b_ref, out_ref):
    out_ref[...] = ((acc_ref[...] + hs_ref[...]) * dinv_ref[...]
                    + b_ref[...])


def _post(acc, hs, dinv, b):
    return pl.pallas_call(
        _post_body,
        grid=(N_PAD // BM,),
        in_specs=[
            pl.BlockSpec((BM, D), lambda i: (i, 0)),
            pl.BlockSpec((BM, D), lambda i: (i, 0)),
            pl.BlockSpec((BM, 1), lambda i: (i, 0)),
            pl.BlockSpec((1, D), lambda i: (0, 0)),
        ],
        out_specs=pl.BlockSpec((BM, D), lambda i: (i, 0)),
        out_shape=jax.ShapeDtypeStruct((N_PAD, D), jnp.float32),
    )(acc, hs, dinv, b)


def kernel(x, edge_index, W0, b0, W1, b1, W2, b2):
    x_pad = jnp.zeros((N_PAD, D), jnp.float32).at[:N].set(x)
    pad = jnp.full((E_PAD - E,), N, jnp.int32)
    src_p = jnp.concatenate([edge_index[0], pad])
    dst_p = jnp.concatenate([edge_index[1], pad])

    degp = _sc_degree_kernel()(dst_p).reshape(NC, N_PAD, D)[:, :, :16]
    hs, dinv = _pre(x_pad, W0, degp)
    acc = _sc_scatter_kernel()(hs, src_p, dst_p)
    hs = _mid(acc, hs, dinv, b0.reshape(1, D), W1)
    acc = _sc_scatter_kernel()(hs, src_p, dst_p)
    hs = _mid(acc, hs, dinv, b1.reshape(1, D), W2)
    acc = _sc_scatter_kernel()(hs, src_p, dst_p)
    out = _post(acc, hs, dinv, b2.reshape(1, D))
    return out[:N]


# speedup vs baseline: 3.9227x; 1.3703x over previous
import functools

import jax
import jax.numpy as jnp
from jax import lax
from jax.experimental import pallas as pl
from jax.experimental.pallas import tpu as pltpu
from jax.experimental.pallas import tpu_sc as plsc

N = 10000
E = 320000
D = 128

NC = 2
NS = 16
CH = 128
NW = NC * NS

N_PAD = 10240
_ITERS_ALL = -(-E // (NS * CH))
_ITERS_ALL += (-_ITERS_ALL) % 4
E_PAD = _ITERS_ALL * NS * CH
EPT = E_PAD // NW
_ITERS_DEG = EPT // CH
ROWS_PT = N_PAD // NS

_SC_PARAMS = dict(use_tc_tiling_on_sc=False)


@functools.cache
def _sc_degree_kernel():
    mesh = plsc.VectorSubcoreMesh(core_axis_name="c", subcore_axis_name="s")
    return pl.kernel(
        _sc_degree_body,
        out_type=jax.ShapeDtypeStruct((NC * N_PAD, D), jnp.float32),
        mesh=mesh,
        compiler_params=pltpu.CompilerParams(**_SC_PARAMS),
        scratch_types=[
            pltpu.VMEM((CH,), jnp.int32),
            pltpu.VMEM((CH,), jnp.int32),
            pltpu.VMEM((CH, 16), jnp.float32),
            pltpu.VMEM((ROWS_PT, 16), jnp.float32),
            pltpu.VMEM_SHARED((N_PAD, 16), jnp.float32),
            pltpu.SemaphoreType.DMA,
            pltpu.SemaphoreType.DMA,
        ],
    )


def _sc_degree_body(dst_hbm, out_hbm, dst0, dst1, ones_v, zero_v, deg_sh,
                    isem0, isem1):
    c = lax.axis_index("c")
    s = lax.axis_index("s")
    base = (c * NS + s) * _ITERS_DEG

    def _fill(i, _):
        ones_v[i] = jnp.ones((16,), jnp.float32)
        return _
    lax.fori_loop(0, CH, _fill, None)

    def _zero(i, _):
        zero_v[i] = jnp.zeros((16,), jnp.float32)
        return _
    lax.fori_loop(0, ROWS_PT, _zero, None)

    pltpu.sync_copy(zero_v, deg_sh.at[pl.ds(s * ROWS_PT, ROWS_PT)])
    plsc.subcore_barrier()

    def _idx_start(j, buf, isem):
        pltpu.async_copy(dst_hbm.at[pl.ds((base + j) * CH, CH)], buf, isem)

    def _idx_wait(j, buf, isem):
        pltpu.make_async_copy(dst_hbm.at[pl.ds((base + j) * CH, CH)], buf,
                              isem).wait()

    _idx_start(0, dst0, isem0)

    def _step(e, _):
        def _one(j, buf, isem, nbuf, nisem):
            _idx_wait(j, buf, isem)

            @pl.when(j + 1 < _ITERS_DEG)
            def _():
                _idx_start(j + 1, nbuf, nisem)
            pltpu.sync_copy(ones_v, deg_sh.at[buf], add=True)

        _one(2 * e, dst0, isem0, dst1, isem1)

        @pl.when(2 * e + 1 < _ITERS_DEG)
        def _():
            _one(2 * e + 1, dst1, isem1, dst0, isem0)
        return _
    lax.fori_loop(0, (_ITERS_DEG + 1) // 2, _step, None)

    plsc.subcore_barrier()
    pltpu.sync_copy(deg_sh.at[pl.ds(s * ROWS_PT, ROWS_PT)],
                    out_hbm.at[pl.ds(c * N_PAD + s * ROWS_PT, ROWS_PT),
                               pl.ds(0, 16)])


DH = D // NC
ITERS_ALL = E_PAD // CH // NS


@functools.cache
def _sc_scatter_kernel():
    mesh = plsc.VectorSubcoreMesh(core_axis_name="c", subcore_axis_name="s")
    return pl.kernel(
        _sc_scatter_body,
        out_type=jax.ShapeDtypeStruct((N_PAD, D), jnp.float32),
        mesh=mesh,
        compiler_params=pltpu.CompilerParams(**_SC_PARAMS),
        scratch_types=(
            [pltpu.VMEM((CH,), jnp.int32)] * 8
            + [pltpu.VMEM((CH, DH), jnp.float32)] * 4
            + [
                pltpu.VMEM_SHARED((N_PAD, DH), jnp.float32),
                pltpu.VMEM_SHARED((N_PAD, DH), jnp.float32),
            ]
            + [pltpu.SemaphoreType.DMA] * 12
        ),
    )


def _sc_scatter_body(hs_hbm, src_hbm, dst_hbm, out_hbm,
                     src0, dst0, src1, dst1, src2, dst2, src3, dst3,
                     rows0, rows1, rows2, rows3, hs_sh, acc_sh,
                     g0, g1, g2, g3, s0, s1, s2, s3, i0, i1, i2, i3):
    c = lax.axis_index("c")
    s = lax.axis_index("s")
    base = s * ITERS_ALL
    SRC = [src0, src1, src2, src3]
    DST = [dst0, dst1, dst2, dst3]
    ROWS = [rows0, rows1, rows2, rows3]
    GSEM = [g0, g1, g2, g3]
    SSEM = [s0, s1, s2, s3]
    ISEM = [i0, i1, i2, i3]

    pltpu.sync_copy(hs_hbm.at[pl.ds(s * ROWS_PT, ROWS_PT),
                              pl.ds(c * DH, DH)],
                    hs_sh.at[pl.ds(s * ROWS_PT, ROWS_PT)])

    def _zrow(r, _):
        def _zcol(k, __):
            rows0[r, pl.ds(k * 16, 16)] = jnp.zeros((16,), jnp.float32)
            return __
        lax.fori_loop(0, DH // 16, _zcol, None)
        return _
    lax.fori_loop(0, CH, _zrow, None)

    def _zacc(j, _):
        pltpu.sync_copy(rows0, acc_sh.at[pl.ds(s * ROWS_PT + j * CH, CH)])
        return _
    lax.fori_loop(0, ROWS_PT // CH, _zacc, None)
    plsc.subcore_barrier()

    def _idx_start(j, t):
        off = (base + j) * CH
        pltpu.async_copy(src_hbm.at[pl.ds(off, CH)], SRC[t], ISEM[t])
        pltpu.async_copy(dst_hbm.at[pl.ds(off, CH)], DST[t], ISEM[t])

    def _idx_wait(j, t):
        off = (base + j) * CH
        pltpu.make_async_copy(src_hbm.at[pl.ds(off, CH)], SRC[t],
                              ISEM[t]).wait()
        pltpu.make_async_copy(dst_hbm.at[pl.ds(off, CH)], DST[t],
                              ISEM[t]).wait()

    def _scat_wait(t):
        pltpu.make_async_copy(ROWS[t], acc_sh.at[DST[t]], SSEM[t]).wait()

    _idx_start(0, 0)
    _idx_start(1, 1)
    _idx_wait(0, 0)
    pltpu.async_copy(hs_sh.at[src0], rows0, GSEM[0])

    def _step(j, t):
        X, Y, Z = t % 4, (t + 1) % 4, (t + 2) % 4
        pltpu.make_async_copy(hs_sh.at[SRC[X]], ROWS[X], GSEM[X]).wait()

        @pl.when(j + 1 < ITERS_ALL)
        def _():
            _idx_wait(j + 1, Y)

        @pl.when(j >= 2)
        def _():
            _scat_wait(Z)

        @pl.when(j + 1 < ITERS_ALL)
        def _():
            pltpu.async_copy(hs_sh.at[SRC[Y]], ROWS[Y], GSEM[Y])
        pltpu.async_copy(ROWS[X], acc_sh.at[DST[X]], SSEM[X], add=True)

        @pl.when(j + 2 < ITERS_ALL)
        def _():
            _idx_start(j + 2, Z)

    def _quad(k, _):
        j = 4 * k
        for t in range(4):
            _step(j + t, t)
        return _
    lax.fori_loop(0, ITERS_ALL // 4, _quad, None)

    _scat_wait((ITERS_ALL - 2) % 4)
    _scat_wait((ITERS_ALL - 1) % 4)
    plsc.subcore_barrier()
    pltpu.sync_copy(acc_sh.at[pl.ds(s * ROWS_PT, ROWS_PT)],
                    out_hbm.at[pl.ds(s * ROWS_PT, ROWS_PT),
                               pl.ds(c * DH, DH)])


BM = 1024


def _pre_body(x_ref, w_ref, degp_ref, hs_ref, dinv_ref):
    deg = (jnp.sum(degp_ref[0], axis=1)
           + jnp.sum(degp_ref[1], axis=1)) * (1.0 / 16.0) + 1.0
    dinv = lax.rsqrt(deg)
    h = jnp.dot(x_ref[...], w_ref[...], preferred_element_type=jnp.float32)
    hs_ref[...] = h * dinv[:, None]
    dinv_ref[...] = dinv[:, None]


def _pre(x_pad, w0, degp):
    return pl.pallas_call(
        _pre_body,
        grid=(N_PAD // BM,),
        in_specs=[
            pl.BlockSpec((BM, D), lambda i: (i, 0)),
            pl.BlockSpec((D, D), lambda i: (0, 0)),
            pl.BlockSpec((NC, BM, 16), lambda i: (0, i, 0)),
        ],
        out_specs=[
            pl.BlockSpec((BM, D), lambda i: (i, 0)),
            pl.BlockSpec((BM, 1), lambda i: (i, 0)),
        ],
        out_shape=[
            jax.ShapeDtypeStruct((N_PAD, D), jnp.float32),
            jax.ShapeDtypeStruct((N_PAD, 1), jnp.float32),
        ],
    )(x_pad, w0, degp)


def _mid_body(acc_ref, hs_ref, dinv_ref, b_ref, w_ref, out_ref):
    dinv = dinv_ref[...]
    o = (acc_ref[...] + hs_ref[...]) * dinv + b_ref[...]
    x = jnp.maximum(o, 0.0)
    out_ref[...] = jnp.dot(x, w_ref[...],
                           preferred_element_type=jnp.float32) * dinv


def _mid(acc, hs, dinv, b, w_next):
    return pl.pallas_call(
        _mid_body,
        grid=(N_PAD // BM,),
        in_specs=[
            pl.BlockSpec((BM, D), lambda i: (i, 0)),
            pl.BlockSpec((BM, D), lambda i: (i, 0)),
            pl.BlockSpec((BM, 1), lambda i: (i, 0)),
            pl.BlockSpec((1, D), lambda i: (0, 0)),
            pl.BlockSpec((D, D), lambda i: (0, 0)),
        ],
        out_specs=pl.BlockSpec((BM, D), lambda i: (i, 0)),
        out_shape=jax.ShapeDtypeStruct((N_PAD, D), jnp.float32),
    )(acc, hs, dinv, b, w_next)


def _post_body(acc_ref, hs_ref, dinv_ref, b_ref, out_ref):
    out_ref[...] = ((acc_ref[...] + hs_ref[...]) * dinv_ref[...]
                    + b_ref[...])


def _post(acc, hs, dinv, b):
    return pl.pallas_call(
        _post_body,
        grid=(N_PAD // BM,),
        in_specs=[
            pl.BlockSpec((BM, D), lambda i: (i, 0)),
            pl.BlockSpec((BM, D), lambda i: (i, 0)),
            pl.BlockSpec((BM, 1), lambda i: (i, 0)),
            pl.BlockSpec((1, D), lambda i: (0, 0)),
        ],
        out_specs=pl.BlockSpec((BM, D), lambda i: (i, 0)),
        out_shape=jax.ShapeDtypeStruct((N_PAD, D), jnp.float32),
    )(acc, hs, dinv, b)


def kernel(x, edge_index, W0, b0, W1, b1, W2, b2):
    x_pad = jnp.zeros((N_PAD, D), jnp.float32).at[:N].set(x)
    pad = jnp.full((E_PAD - E,), N, jnp.int32)
    src_p = jnp.concatenate([edge_index[0], pad])
    dst_p = jnp.concatenate([edge_index[1], pad])

    degp = _sc_degree_kernel()(dst_p).reshape(NC, N_PAD, D)[:, :, :16]
    hs, dinv = _pre(x_pad, W0, degp)
    acc = _sc_scatter_kernel()(hs, src_p, dst_p)
    hs = _mid(acc, hs, dinv, b0.reshape(1, D), W1)
    acc = _sc_scatter_kernel()(hs, src_p, dst_p)
    hs = _mid(acc, hs, dinv, b1.reshape(1, D), W2)
    acc = _sc_scatter_kernel()(hs, src_p, dst_p)
    out = _post(acc, hs, dinv, b2.reshape(1, D))
    return out[:N]


# speedup vs baseline: 4.0596x; 1.0349x over previous
import functools

import jax
import jax.numpy as jnp
from jax import lax
from jax.experimental import pallas as pl
from jax.experimental.pallas import tpu as pltpu
from jax.experimental.pallas import tpu_sc as plsc

N = 10000
E = 320000
D = 128

NC = 2
NS = 16
CH = 128
NW = NC * NS

N_PAD = 10240
_ITERS_ALL = -(-E // (NS * CH))
_ITERS_ALL += (-_ITERS_ALL) % 4
E_PAD = _ITERS_ALL * NS * CH
EPT = E_PAD // NW
_ITERS_DEG = EPT // CH
ROWS_PT = N_PAD // NS

_SC_PARAMS = dict(use_tc_tiling_on_sc=False)


@functools.cache
def _sc_degree_kernel():
    mesh = plsc.VectorSubcoreMesh(core_axis_name="c", subcore_axis_name="s")
    return pl.kernel(
        _sc_degree_body,
        out_type=jax.ShapeDtypeStruct((NC * N_PAD, D), jnp.float32),
        mesh=mesh,
        compiler_params=pltpu.CompilerParams(**_SC_PARAMS),
        scratch_types=(
            [pltpu.VMEM((CH,), jnp.int32)] * 4
            + [
                pltpu.VMEM((CH, 16), jnp.float32),
                pltpu.VMEM((ROWS_PT, 16), jnp.float32),
                pltpu.VMEM_SHARED((N_PAD, 16), jnp.float32),
            ]
            + [pltpu.SemaphoreType.DMA] * 8
        ),
    )


def _sc_degree_body(dst_hbm, out_hbm, dst0, dst1, dst2, dst3, ones_v,
                    zero_v, deg_sh, i0, i1, i2, i3, s0, s1, s2, s3):
    c = lax.axis_index("c")
    s = lax.axis_index("s")
    base = (c * NS + s) * _ITERS_DEG
    DST = [dst0, dst1, dst2, dst3]
    ISEM = [i0, i1, i2, i3]
    SSEM = [s0, s1, s2, s3]

    def _fill(i, _):
        ones_v[i] = jnp.ones((16,), jnp.float32)
        return _
    lax.fori_loop(0, CH, _fill, None)

    def _zero(i, _):
        zero_v[i] = jnp.zeros((16,), jnp.float32)
        return _
    lax.fori_loop(0, ROWS_PT, _zero, None)

    pltpu.sync_copy(zero_v, deg_sh.at[pl.ds(s * ROWS_PT, ROWS_PT)])
    plsc.subcore_barrier()

    def _idx_start(j, t):
        pltpu.async_copy(dst_hbm.at[pl.ds((base + j) * CH, CH)], DST[t],
                         ISEM[t])

    def _idx_wait(j, t):
        pltpu.make_async_copy(dst_hbm.at[pl.ds((base + j) * CH, CH)],
                              DST[t], ISEM[t]).wait()

    def _scat_wait(t):
        pltpu.make_async_copy(ones_v, deg_sh.at[DST[t]], SSEM[t]).wait()

    _idx_start(0, 0)
    _idx_start(1, 1)

    def _step(j, t):
        X, Z = t % 4, (t + 2) % 4
        _idx_wait(j, X)

        @pl.when(j >= 2)
        def _():
            _scat_wait(Z)
        pltpu.async_copy(ones_v, deg_sh.at[DST[X]], SSEM[X], add=True)

        @pl.when(j + 2 < _ITERS_DEG)
        def _():
            _idx_start(j + 2, Z)

    def _quad(k, _):
        j = 4 * k
        for t in range(4):
            _step(j + t, t)
        return _
    lax.fori_loop(0, _ITERS_DEG // 4, _quad, None)

    _scat_wait((_ITERS_DEG - 2) % 4)
    _scat_wait((_ITERS_DEG - 1) % 4)
    plsc.subcore_barrier()
    pltpu.sync_copy(deg_sh.at[pl.ds(s * ROWS_PT, ROWS_PT)],
                    out_hbm.at[pl.ds(c * N_PAD + s * ROWS_PT, ROWS_PT),
                               pl.ds(0, 16)])


DH = D // NC
ITERS_ALL = E_PAD // CH // NS


@functools.cache
def _sc_scatter_kernel():
    mesh = plsc.VectorSubcoreMesh(core_axis_name="c", subcore_axis_name="s")
    return pl.kernel(
        _sc_scatter_body,
        out_type=jax.ShapeDtypeStruct((N_PAD, D), jnp.float32),
        mesh=mesh,
        compiler_params=pltpu.CompilerParams(**_SC_PARAMS),
        scratch_types=(
            [pltpu.VMEM((CH,), jnp.int32)] * 8
            + [pltpu.VMEM((CH, DH), jnp.float32)] * 4
            + [
                pltpu.VMEM_SHARED((N_PAD, DH), jnp.float32),
                pltpu.VMEM_SHARED((N_PAD, DH), jnp.float32),
            ]
            + [pltpu.SemaphoreType.DMA] * 12
        ),
    )


def _sc_scatter_body(hs_hbm, src_hbm, dst_hbm, out_hbm,
                     src0, dst0, src1, dst1, src2, dst2, src3, dst3,
                     rows0, rows1, rows2, rows3, hs_sh, acc_sh,
                     g0, g1, g2, g3, s0, s1, s2, s3, i0, i1, i2, i3):
    c = lax.axis_index("c")
    s = lax.axis_index("s")
    base = s * ITERS_ALL
    SRC = [src0, src1, src2, src3]
    DST = [dst0, dst1, dst2, dst3]
    ROWS = [rows0, rows1, rows2, rows3]
    GSEM = [g0, g1, g2, g3]
    SSEM = [s0, s1, s2, s3]
    ISEM = [i0, i1, i2, i3]

    pltpu.sync_copy(hs_hbm.at[pl.ds(s * ROWS_PT, ROWS_PT),
                              pl.ds(c * DH, DH)],
                    hs_sh.at[pl.ds(s * ROWS_PT, ROWS_PT)])

    def _zrow(r, _):
        def _zcol(k, __):
            rows0[r, pl.ds(k * 16, 16)] = jnp.zeros((16,), jnp.float32)
            return __
        lax.fori_loop(0, DH // 16, _zcol, None)
        return _
    lax.fori_loop(0, CH, _zrow, None)

    def _zacc(j, _):
        pltpu.sync_copy(rows0, acc_sh.at[pl.ds(s * ROWS_PT + j * CH, CH)])
        return _
    lax.fori_loop(0, ROWS_PT // CH, _zacc, None)
    plsc.subcore_barrier()

    def _idx_start(j, t):
        off = (base + j) * CH
        pltpu.async_copy(src_hbm.at[pl.ds(off, CH)], SRC[t], ISEM[t])
        pltpu.async_copy(dst_hbm.at[pl.ds(off, CH)], DST[t], ISEM[t])

    def _idx_wait(j, t):
        off = (base + j) * CH
        pltpu.make_async_copy(src_hbm.at[pl.ds(off, CH)], SRC[t],
                              ISEM[t]).wait()
        pltpu.make_async_copy(dst_hbm.at[pl.ds(off, CH)], DST[t],
                              ISEM[t]).wait()

    def _scat_wait(t):
        pltpu.make_async_copy(ROWS[t], acc_sh.at[DST[t]], SSEM[t]).wait()

    _idx_start(0, 0)
    _idx_start(1, 1)
    _idx_wait(0, 0)
    pltpu.async_copy(hs_sh.at[src0], rows0, GSEM[0])

    def _step(j, t):
        X, Y, Z = t % 4, (t + 1) % 4, (t + 2) % 4
        pltpu.make_async_copy(hs_sh.at[SRC[X]], ROWS[X], GSEM[X]).wait()

        @pl.when(j + 1 < ITERS_ALL)
        def _():
            _idx_wait(j + 1, Y)

        @pl.when(j >= 2)
        def _():
            _scat_wait(Z)

        @pl.when(j + 1 < ITERS_ALL)
        def _():
            pltpu.async_copy(hs_sh.at[SRC[Y]], ROWS[Y], GSEM[Y])
        pltpu.async_copy(ROWS[X], acc_sh.at[DST[X]], SSEM[X], add=True)

        @pl.when(j + 2 < ITERS_ALL)
        def _():
            _idx_start(j + 2, Z)

    def _quad(k, _):
        j = 4 * k
        for t in range(4):
            _step(j + t, t)
        return _
    lax.fori_loop(0, ITERS_ALL // 4, _quad, None)

    _scat_wait((ITERS_ALL - 2) % 4)
    _scat_wait((ITERS_ALL - 1) % 4)
    plsc.subcore_barrier()
    pltpu.sync_copy(acc_sh.at[pl.ds(s * ROWS_PT, ROWS_PT)],
                    out_hbm.at[pl.ds(s * ROWS_PT, ROWS_PT),
                               pl.ds(c * DH, DH)])


BM = 1024


def _pre_body(x_ref, w_ref, degp_ref, hs_ref, dinv_ref):
    deg = (jnp.sum(degp_ref[0], axis=1)
           + jnp.sum(degp_ref[1], axis=1)) * (1.0 / 16.0) + 1.0
    dinv = lax.rsqrt(deg)
    h = jnp.dot(x_ref[...], w_ref[...], preferred_element_type=jnp.float32)
    hs_ref[...] = h * dinv[:, None]
    dinv_ref[...] = dinv[:, None]


def _pre(x_pad, w0, degp):
    return pl.pallas_call(
        _pre_body,
        grid=(N_PAD // BM,),
        in_specs=[
            pl.BlockSpec((BM, D), lambda i: (i, 0)),
            pl.BlockSpec((D, D), lambda i: (0, 0)),
            pl.BlockSpec((NC, BM, 16), lambda i: (0, i, 0)),
        ],
        out_specs=[
            pl.BlockSpec((BM, D), lambda i: (i, 0)),
            pl.BlockSpec((BM, 1), lambda i: (i, 0)),
        ],
        out_shape=[
            jax.ShapeDtypeStruct((N_PAD, D), jnp.float32),
            jax.ShapeDtypeStruct((N_PAD, 1), jnp.float32),
        ],
    )(x_pad, w0, degp)


def _mid_body(acc_ref, hs_ref, dinv_ref, b_ref, w_ref, out_ref):
    dinv = dinv_ref[...]
    o = (acc_ref[...] + hs_ref[...]) * dinv + b_ref[...]
    x = jnp.maximum(o, 0.0)
    out_ref[...] = jnp.dot(x, w_ref[...],
                           preferred_element_type=jnp.float32) * dinv


def _mid(acc, hs, dinv, b, w_next):
    return pl.pallas_call(
        _mid_body,
        grid=(N_PAD // BM,),
        in_specs=[
            pl.BlockSpec((BM, D), lambda i: (i, 0)),
            pl.BlockSpec((BM, D), lambda i: (i, 0)),
            pl.BlockSpec((BM, 1), lambda i: (i, 0)),
            pl.BlockSpec((1, D), lambda i: (0, 0)),
            pl.BlockSpec((D, D), lambda i: (0, 0)),
        ],
        out_specs=pl.BlockSpec((BM, D), lambda i: (i, 0)),
        out_shape=jax.ShapeDtypeStruct((N_PAD, D), jnp.float32),
    )(acc, hs, dinv, b, w_next)


def _post_body(acc_ref, hs_ref, dinv_ref, b_ref, out_ref):
    out_ref[...] = ((acc_ref[...] + hs_ref[...]) * dinv_ref[...]
                    + b_ref[...])


def _post(acc, hs, dinv, b):
    return pl.pallas_call(
        _post_body,
        grid=(N_PAD // BM,),
        in_specs=[
            pl.BlockSpec((BM, D), lambda i: (i, 0)),
            pl.BlockSpec((BM, D), lambda i: (i, 0)),
            pl.BlockSpec((BM, 1), lambda i: (i, 0)),
            pl.BlockSpec((1, D), lambda i: (0, 0)),
        ],
        out_specs=pl.BlockSpec((BM, D), lambda i: (i, 0)),
        out_shape=jax.ShapeDtypeStruct((N_PAD, D), jnp.float32),
    )(acc, hs, dinv, b)


def kernel(x, edge_index, W0, b0, W1, b1, W2, b2):
    x_pad = jnp.zeros((N_PAD, D), jnp.float32).at[:N].set(x)
    pad = jnp.full((E_PAD - E,), N, jnp.int32)
    src_p = jnp.concatenate([edge_index[0], pad])
    dst_p = jnp.concatenate([edge_index[1], pad])

    degp = _sc_degree_kernel()(dst_p).reshape(NC, N_PAD, D)[:, :, :16]
    hs, dinv = _pre(x_pad, W0, degp)
    acc = _sc_scatter_kernel()(hs, src_p, dst_p)
    hs = _mid(acc, hs, dinv, b0.reshape(1, D), W1)
    acc = _sc_scatter_kernel()(hs, src_p, dst_p)
    hs = _mid(acc, hs, dinv, b1.reshape(1, D), W2)
    acc = _sc_scatter_kernel()(hs, src_p, dst_p)
    out = _post(acc, hs, dinv, b2.reshape(1, D))
    return out[:N]
